# baseline (speedup 1.0000x reference)
import functools

import jax
import jax.numpy as jnp
from jax import lax
from jax.experimental import pallas as pl
from jax.experimental.pallas import tpu as pltpu
from jax.experimental.pallas import tpu_sc as plsc

NC = 2
NS = 16
NW = NC * NS
L = 16

N_FLOATS = 16384 * 50 * 2
PER_TILE = N_FLOATS // NW
N_VECS = PER_TILE // L

_MAGIC = 0x5F3759DF


def _sc_body(pred_hbm, gt_hbm, loss_hbm, cnt_hbm, p_v, g_v, out_v, cnt_v):
    wid = lax.axis_index("s") * NC + lax.axis_index("c")
    base = wid * PER_TILE

    pltpu.sync_copy(pred_hbm.at[pl.ds(base, PER_TILE)], p_v)
    pltpu.sync_copy(gt_hbm.at[pl.ds(base, PER_TILE)], g_v)

    lane = lax.iota(jnp.int32, L)
    swap_idx = lane ^ 1
    parity = (lane & 1) == 0
    dnums = lax.GatherDimensionNumbers(
        offset_dims=(), collapsed_slice_dims=(0,), start_index_map=(0,))

    def one_vec(g, p, acc_l, acc_c):
        e = g - p
        s = e * e
        sw = lax.gather(
            s, swap_idx[:, None], dnums, slice_sizes=(1,),
            mode=lax.GatherScatterMode.PROMISE_IN_BOUNDS)
        rs = s + sw
        yi = jnp.int32(_MAGIC) - (lax.bitcast_convert_type(rs, jnp.int32) >> 1)
        y = lax.bitcast_convert_type(yi, jnp.float32)
        y = y * (1.5 - (rs * 0.5) * y * y)
        norm = rs * y
        m = (g != 0.0) & parity
        acc_l = acc_l + jnp.where(m, norm, 0.0)
        acc_c = acc_c + jnp.where(m, jnp.int32(1), jnp.int32(0))
        return acc_l, acc_c

    U = 8

    def body(i, carry):
        accs_l, accs_c = carry
        base_i = i * (U * L)
        new_l, new_c = [], []
        for u in range(U):
            g = g_v[pl.ds(base_i + u * L, L)]
            p = p_v[pl.ds(base_i + u * L, L)]
            al, ac = one_vec(g, p, accs_l[u], accs_c[u])
            new_l.append(al)
            new_c.append(ac)
        return tuple(new_l), tuple(new_c)

    zero_l = tuple(jnp.zeros((L,), jnp.float32) for _ in range(U))
    zero_c = tuple(jnp.zeros((L,), jnp.int32) for _ in range(U))
    accs_l, accs_c = lax.fori_loop(0, N_VECS // U, body, (zero_l, zero_c))

    acc_l = functools.reduce(lambda a, b: a + b, accs_l)
    acc_c = functools.reduce(lambda a, b: a + b, accs_c)

    out_v[...] = acc_l
    cnt_v[...] = acc_c
    pltpu.sync_copy(out_v, loss_hbm.at[wid])
    pltpu.sync_copy(cnt_v, cnt_hbm.at[wid])


@jax.jit
def kernel(pred_out, pred_gt):
    pred_flat = pred_out.reshape(-1)
    gt_flat = pred_gt.reshape(-1)
    mesh = plsc.VectorSubcoreMesh(
        core_axis_name="c", subcore_axis_name="s", num_cores=NC,
        num_subcores=NS)
    loss, cnt = pl.kernel(
        _sc_body,
        out_type=[
            jax.ShapeDtypeStruct((NW, L), jnp.float32),
            jax.ShapeDtypeStruct((NW, L), jnp.int32),
        ],
        mesh=mesh,
        scratch_types=[
            pltpu.VMEM((PER_TILE,), jnp.float32),
            pltpu.VMEM((PER_TILE,), jnp.float32),
            pltpu.VMEM((L,), jnp.float32),
            pltpu.VMEM((L,), jnp.int32),
        ],
    )(pred_flat, gt_flat)
    return jnp.sum(loss), jnp.sum(cnt)


# speedup vs baseline: 1.0003x; 1.0003x over previous
import functools

import jax
import jax.numpy as jnp
from jax import lax
from jax.experimental import pallas as pl
from jax.experimental.pallas import tpu as pltpu
from jax.experimental.pallas import tpu_sc as plsc

NC = 2
NS = 16
NW = NC * NS
L = 16

N_FLOATS = 16384 * 50 * 2
PER_TILE = N_FLOATS // NW
N_VECS = PER_TILE // L

_MAGIC = 0x5F3759DF


ROWS_PER_TILE = PER_TILE // 128


def _sc_body(pred_hbm2, gt_hbm2, loss_hbm, cnt_hbm, p_v, g_v, out_v, cnt_v):
    wid = lax.axis_index("s") * NC + lax.axis_index("c")
    row0 = wid * ROWS_PER_TILE

    pltpu.sync_copy(pred_hbm2.at[pl.ds(row0, ROWS_PER_TILE)], p_v)
    pltpu.sync_copy(gt_hbm2.at[pl.ds(row0, ROWS_PER_TILE)], g_v)

    lane = lax.iota(jnp.int32, L)
    swap_idx = lane ^ 1
    parity = (lane & 1) == 0
    dnums = lax.GatherDimensionNumbers(
        offset_dims=(), collapsed_slice_dims=(0,), start_index_map=(0,))

    def one_vec(g, p, acc_l, acc_c):
        e = g - p
        s = e * e
        sw = lax.gather(
            s, swap_idx[:, None], dnums, slice_sizes=(1,),
            mode=lax.GatherScatterMode.PROMISE_IN_BOUNDS)
        rs = s + sw
        yi = jnp.int32(_MAGIC) - (lax.bitcast_convert_type(rs, jnp.int32) >> 1)
        y = lax.bitcast_convert_type(yi, jnp.float32)
        y = y * (1.5 - (rs * 0.5) * y * y)
        norm = rs * y
        m = (g != 0.0) & parity
        acc_l = acc_l + jnp.where(m, norm, 0.0)
        acc_c = acc_c + jnp.where(m, jnp.int32(1), jnp.int32(0))
        return acc_l, acc_c

    U = 8

    def body(r, carry):
        accs_l, accs_c = carry
        new_l, new_c = [], []
        for u in range(U):
            g = g_v[r, pl.ds(u * L, L)]
            p = p_v[r, pl.ds(u * L, L)]
            al, ac = one_vec(g, p, accs_l[u], accs_c[u])
            new_l.append(al)
            new_c.append(ac)
        return tuple(new_l), tuple(new_c)

    zero_l = tuple(jnp.zeros((L,), jnp.float32) for _ in range(U))
    zero_c = tuple(jnp.zeros((L,), jnp.int32) for _ in range(U))
    accs_l, accs_c = lax.fori_loop(0, ROWS_PER_TILE, body, (zero_l, zero_c))

    acc_l = functools.reduce(lambda a, b: a + b, accs_l)
    acc_c = functools.reduce(lambda a, b: a + b, accs_c)

    out_v[...] = acc_l
    cnt_v[...] = acc_c
    pltpu.sync_copy(out_v, loss_hbm.at[wid])
    pltpu.sync_copy(cnt_v, cnt_hbm.at[wid])


@jax.jit
def kernel(pred_out, pred_gt):
    pred_flat = pred_out.reshape(12800, 128)
    gt_flat = pred_gt.reshape(12800, 128)
    mesh = plsc.VectorSubcoreMesh(
        core_axis_name="c", subcore_axis_name="s", num_cores=NC,
        num_subcores=NS)
    loss, cnt = pl.kernel(
        _sc_body,
        out_type=[
            jax.ShapeDtypeStruct((NW, L), jnp.float32),
            jax.ShapeDtypeStruct((NW, L), jnp.int32),
        ],
        mesh=mesh,
        scratch_types=[
            pltpu.VMEM((ROWS_PER_TILE, 128), jnp.float32),
            pltpu.VMEM((ROWS_PER_TILE, 128), jnp.float32),
            pltpu.VMEM((L,), jnp.float32),
            pltpu.VMEM((L,), jnp.int32),
        ],
    )(pred_flat, gt_flat)
    return jnp.sum(loss), jnp.sum(cnt)


# speedup vs baseline: 1.0050x; 1.0047x over previous
import jax
import jax.numpy as jnp
from jax import lax
from jax.experimental import pallas as pl
from jax.experimental.pallas import tpu as pltpu

NROW = 12800
NCOL = 128
BR = 1600
GRID = NROW // BR


def _tc_body(p_ref, g_ref, loss_ref, cnt_ref):
    step = pl.program_id(0)

    g = g_ref[...]
    p = p_ref[...]
    e = g - p
    s = e * e
    rolled = pltpu.roll(s, shift=NCOL - 1, axis=1)
    rs = s + rolled
    norm = jnp.sqrt(rs)
    lane = lax.broadcasted_iota(jnp.int32, (BR, NCOL), 1)
    m = ((lane & 1) == 0) & (g != 0.0)
    part_l = jnp.sum(jnp.where(m, norm, 0.0))
    part_c = jnp.sum(jnp.where(m, 1.0, 0.0))

    @pl.when(step == 0)
    def _init():
        loss_ref[0, 0] = 0.0
        cnt_ref[0, 0] = 0.0

    loss_ref[0, 0] += part_l
    cnt_ref[0, 0] += part_c


@jax.jit
def kernel(pred_out, pred_gt):
    p2 = pred_out.reshape(NROW, NCOL)
    g2 = pred_gt.reshape(NROW, NCOL)
    loss, cnt = pl.pallas_call(
        _tc_body,
        grid=(GRID,),
        in_specs=[
            pl.BlockSpec((BR, NCOL), lambda i: (i, 0)),
            pl.BlockSpec((BR, NCOL), lambda i: (i, 0)),
        ],
        out_specs=[
            pl.BlockSpec((1, 1), lambda i: (0, 0), memory_space=pltpu.SMEM),
            pl.BlockSpec((1, 1), lambda i: (0, 0), memory_space=pltpu.SMEM),
        ],
        out_shape=[
            jax.ShapeDtypeStruct((1, 1), jnp.float32),
            jax.ShapeDtypeStruct((1, 1), jnp.float32),
        ],
    )(p2, g2)
    return loss[0, 0], cnt[0, 0].astype(jnp.int32)


# speedup vs baseline: 142.3019x; 141.5969x over previous
import jax
import jax.numpy as jnp
from jax.experimental import pallas as pl
from jax.experimental.pallas import tpu as pltpu

T = 50
S = 16384
BS = 8192
GRID = S // BS


def _body(p_ref, g_ref, loss_ref, cnt_ref):
    step = pl.program_id(0)

    g = g_ref[...]
    p = p_ref[...]
    e = g - p
    s = e * e
    rs = s[:, 0, :] + s[:, 1, :]
    norm = jnp.sqrt(rs)
    m = g[:, 0, :] != 0.0
    part_l = jnp.sum(jnp.where(m, norm, 0.0))
    part_c = jnp.sum(jnp.where(m, 1.0, 0.0))

    @pl.when(step == 0)
    def _init():
        loss_ref[0, 0] = 0.0
        cnt_ref[0, 0] = 0.0

    loss_ref[0, 0] += part_l
    cnt_ref[0, 0] += part_c


@jax.jit
def kernel(pred_out, pred_gt):
    pt = jnp.transpose(pred_out, (1, 2, 0))
    gt = jnp.transpose(pred_gt, (1, 2, 0))
    loss, cnt = pl.pallas_call(
        _body,
        grid=(GRID,),
        in_specs=[
            pl.BlockSpec((T, 2, BS), lambda i: (0, 0, i)),
            pl.BlockSpec((T, 2, BS), lambda i: (0, 0, i)),
        ],
        out_specs=[
            pl.BlockSpec((1, 1), lambda i: (0, 0), memory_space=pltpu.SMEM),
            pl.BlockSpec((1, 1), lambda i: (0, 0), memory_space=pltpu.SMEM),
        ],
        out_shape=[
            jax.ShapeDtypeStruct((1, 1), jnp.float32),
            jax.ShapeDtypeStruct((1, 1), jnp.float32),
        ],
    )(pt, gt)
    return loss[0, 0], cnt[0, 0].astype(jnp.int32)
